# Initial kernel scaffold; baseline (speedup 1.0000x reference)
import jax
import jax.numpy as jnp
from jax.experimental import pallas as pl


def kernel(hidden_states):
    raise NotImplementedError("write your pallas kernel here")


# speedup vs baseline: 1.5035x; 1.5035x over previous
import functools

import jax
import jax.numpy as jnp
from jax.experimental import pallas as pl
from jax.experimental.pallas import tpu as pltpu

_HIDDEN = 768


def _cdiv(a, b):
    return (a + b - 1) // b


def _round_up(x, m):
    return (x + m - 1) // m * m


def _pool_whole_seq_kernel(x_ref, o_ref, *, inv_s):
    x = x_ref[...]
    o_ref[...] = (jnp.sum(x, axis=1, dtype=jnp.float32) * inv_s).astype(
        o_ref.dtype)


def _pool_streamed_kernel(x_ref, o_ref, acc_ref, *, inv_s, s, ts):
    k = pl.program_id(1)

    @pl.when(k == 0)
    def _():
        acc_ref[...] = jnp.zeros_like(acc_ref)

    x = x_ref[...]
    if s % ts != 0:
        pos = jax.lax.broadcasted_iota(jnp.int32, (ts, _HIDDEN), 0) + k * ts
        x = jnp.where((pos < s)[None], x, jnp.zeros([], x.dtype))
    acc_ref[...] += jnp.sum(x, axis=1, dtype=jnp.float32)

    @pl.when(k == pl.num_programs(1) - 1)
    def _():
        o_ref[...] = (acc_ref[...] * inv_s).astype(o_ref.dtype)


def kernel(hidden_states):
    assert hidden_states.shape[-1] == _HIDDEN
    s = hidden_states.shape[-2]
    dtype = hidden_states.dtype
    itemsize = jnp.dtype(dtype).itemsize
    sub = max(8, 32 // itemsize)

    x = hidden_states.reshape(-1, s, _HIDDEN)
    n = x.shape[0]

    in_budget = 52 * 1024 * 1024
    row_bytes = _HIDDEN * itemsize
    seq_block_bytes = s * row_bytes

    if sub * seq_block_bytes * 2 <= in_budget:
        tn = (in_budget // (2 * seq_block_bytes)) // sub * sub
        tn = max(sub, min(tn, _round_up(n, sub)))
        if n > sub:
            tn = min(tn, _round_up(_cdiv(n, 2), sub))
        grid = (_cdiv(n, tn),)
        return pl.pallas_call(
            functools.partial(_pool_whole_seq_kernel, inv_s=1.0 / s),
            out_shape=jax.ShapeDtypeStruct((n, _HIDDEN), dtype),
            grid=grid,
            in_specs=[pl.BlockSpec((tn, s, _HIDDEN), lambda i: (i, 0, 0))],
            out_specs=pl.BlockSpec((tn, _HIDDEN), lambda i: (i, 0)),
            compiler_params=pltpu.CompilerParams(
                dimension_semantics=("parallel",),
                vmem_limit_bytes=64 << 20),
            cost_estimate=pl.CostEstimate(
                flops=n * s * _HIDDEN + n * _HIDDEN,
                transcendentals=0,
                bytes_accessed=(n * s + n) * _HIDDEN * itemsize),
        )(x)

    tn = sub
    ts = (in_budget // (2 * tn * row_bytes)) // sub * sub
    ts = max(sub, min(ts, _round_up(s, sub)))
    grid = (_cdiv(n, tn), _cdiv(s, ts))
    return pl.pallas_call(
        functools.partial(_pool_streamed_kernel, inv_s=1.0 / s, s=s, ts=ts),
        out_shape=jax.ShapeDtypeStruct((n, _HIDDEN), dtype),
        grid_spec=pltpu.PrefetchScalarGridSpec(
            num_scalar_prefetch=0,
            grid=grid,
            in_specs=[pl.BlockSpec((tn, ts, _HIDDEN), lambda i, k: (i, k, 0))],
            out_specs=pl.BlockSpec((tn, _HIDDEN), lambda i, k: (i, 0)),
            scratch_shapes=[pltpu.VMEM((tn, _HIDDEN), jnp.float32)],
        ),
        compiler_params=pltpu.CompilerParams(
            dimension_semantics=("parallel", "arbitrary"),
            vmem_limit_bytes=64 << 20),
        cost_estimate=pl.CostEstimate(
            flops=n * s * _HIDDEN + n * _HIDDEN,
            transcendentals=0,
            bytes_accessed=(n * s + n) * _HIDDEN * itemsize),
    )(x)


# speedup vs baseline: 1.5158x; 1.0082x over previous
import functools

import jax
import jax.numpy as jnp
from jax.experimental import pallas as pl
from jax.experimental.pallas import tpu as pltpu

_HIDDEN = 768


def _cdiv(a, b):
    return (a + b - 1) // b


def _round_up(x, m):
    return (x + m - 1) // m * m


def _pool_whole_seq_kernel(x_ref, o_ref, *, inv_s):
    x = x_ref[...]
    o_ref[...] = (jnp.sum(x, axis=1, dtype=jnp.float32) * inv_s).astype(
        o_ref.dtype)


def _pool_streamed_kernel(x_ref, o_ref, acc_ref, *, inv_s, s, ts):
    k = pl.program_id(1)

    @pl.when(k == 0)
    def _():
        acc_ref[...] = jnp.zeros_like(acc_ref)

    x = x_ref[...]
    if s % ts != 0:
        pos = jax.lax.broadcasted_iota(jnp.int32, (ts, _HIDDEN), 0) + k * ts
        x = jnp.where((pos < s)[None], x, jnp.zeros([], x.dtype))
    acc_ref[...] += jnp.sum(x, axis=1, dtype=jnp.float32)

    @pl.when(k == pl.num_programs(1) - 1)
    def _():
        o_ref[...] = (acc_ref[...] * inv_s).astype(o_ref.dtype)


def kernel(hidden_states):
    assert hidden_states.shape[-1] == _HIDDEN
    s = hidden_states.shape[-2]
    dtype = hidden_states.dtype
    itemsize = jnp.dtype(dtype).itemsize
    sub = max(8, 32 // itemsize)

    x = hidden_states.reshape(-1, s, _HIDDEN)
    n = x.shape[0]

    in_budget = 52 * 1024 * 1024
    row_bytes = _HIDDEN * itemsize
    seq_block_bytes = s * row_bytes

    if sub * seq_block_bytes * 2 <= in_budget:
        tn = (in_budget // (2 * seq_block_bytes)) // sub * sub
        tn = max(sub, min(tn, _round_up(n, sub), 8))
        if n > sub:
            tn = min(tn, _round_up(_cdiv(n, 2), sub))
        grid = (_cdiv(n, tn),)
        return pl.pallas_call(
            functools.partial(_pool_whole_seq_kernel, inv_s=1.0 / s),
            out_shape=jax.ShapeDtypeStruct((n, _HIDDEN), dtype),
            grid=grid,
            in_specs=[pl.BlockSpec((tn, s, _HIDDEN), lambda i: (i, 0, 0))],
            out_specs=pl.BlockSpec((tn, _HIDDEN), lambda i: (i, 0)),
            compiler_params=pltpu.CompilerParams(
                dimension_semantics=("parallel",),
                vmem_limit_bytes=64 << 20),
            cost_estimate=pl.CostEstimate(
                flops=n * s * _HIDDEN + n * _HIDDEN,
                transcendentals=0,
                bytes_accessed=(n * s + n) * _HIDDEN * itemsize),
        )(x)

    tn = sub
    ts = (in_budget // (2 * tn * row_bytes)) // sub * sub
    ts = max(sub, min(ts, _round_up(s, sub)))
    grid = (_cdiv(n, tn), _cdiv(s, ts))
    return pl.pallas_call(
        functools.partial(_pool_streamed_kernel, inv_s=1.0 / s, s=s, ts=ts),
        out_shape=jax.ShapeDtypeStruct((n, _HIDDEN), dtype),
        grid_spec=pltpu.PrefetchScalarGridSpec(
            num_scalar_prefetch=0,
            grid=grid,
            in_specs=[pl.BlockSpec((tn, ts, _HIDDEN), lambda i, k: (i, k, 0))],
            out_specs=pl.BlockSpec((tn, _HIDDEN), lambda i, k: (i, 0)),
            scratch_shapes=[pltpu.VMEM((tn, _HIDDEN), jnp.float32)],
        ),
        compiler_params=pltpu.CompilerParams(
            dimension_semantics=("parallel", "arbitrary"),
            vmem_limit_bytes=64 << 20),
        cost_estimate=pl.CostEstimate(
            flops=n * s * _HIDDEN + n * _HIDDEN,
            transcendentals=0,
            bytes_accessed=(n * s + n) * _HIDDEN * itemsize),
    )(x)


# speedup vs baseline: 1.5173x; 1.0010x over previous
import functools

import jax
import jax.numpy as jnp
from jax.experimental import pallas as pl
from jax.experimental.pallas import tpu as pltpu

_HIDDEN = 768


def _cdiv(a, b):
    return (a + b - 1) // b


def _round_up(x, m):
    return (x + m - 1) // m * m


def _pool_whole_seq_kernel(x_ref, o_ref, *, inv_s):
    x = x_ref[...]
    o_ref[...] = (jnp.sum(x, axis=1, dtype=jnp.float32) * inv_s).astype(
        o_ref.dtype)


def _pool_streamed_kernel(x_ref, o_ref, acc_ref, *, inv_s, s, ts):
    k = pl.program_id(1)

    @pl.when(k == 0)
    def _():
        acc_ref[...] = jnp.zeros_like(acc_ref)

    x = x_ref[...]
    if s % ts != 0:
        pos = jax.lax.broadcasted_iota(jnp.int32, (ts, _HIDDEN), 0) + k * ts
        x = jnp.where((pos < s)[None], x, jnp.zeros([], x.dtype))
    acc_ref[...] += jnp.sum(x, axis=1, dtype=jnp.float32)

    @pl.when(k == pl.num_programs(1) - 1)
    def _():
        o_ref[...] = (acc_ref[...] * inv_s).astype(o_ref.dtype)


def kernel(hidden_states):
    assert hidden_states.shape[-1] == _HIDDEN
    s = hidden_states.shape[-2]
    dtype = hidden_states.dtype
    itemsize = jnp.dtype(dtype).itemsize
    sub = max(8, 32 // itemsize)

    x = hidden_states.reshape(-1, s, _HIDDEN)
    n = x.shape[0]

    in_budget = 52 * 1024 * 1024
    row_bytes = _HIDDEN * itemsize
    seq_block_bytes = s * row_bytes

    if sub * seq_block_bytes * 2 <= in_budget:
        tn = (in_budget // (2 * seq_block_bytes)) // sub * sub
        tn = max(sub, min(tn, _round_up(n, sub), 4))
        in_spec = pl.BlockSpec((tn, s, _HIDDEN), lambda i: (i, 0, 0))
        if n > sub:
            tn = min(tn, _round_up(_cdiv(n, 2), sub))
        grid = (_cdiv(n, tn),)
        return pl.pallas_call(
            functools.partial(_pool_whole_seq_kernel, inv_s=1.0 / s),
            out_shape=jax.ShapeDtypeStruct((n, _HIDDEN), dtype),
            grid=grid,
            in_specs=[in_spec],
            out_specs=pl.BlockSpec((tn, _HIDDEN), lambda i: (i, 0)),
            compiler_params=pltpu.CompilerParams(
                dimension_semantics=("parallel",),
                vmem_limit_bytes=64 << 20),
            cost_estimate=pl.CostEstimate(
                flops=n * s * _HIDDEN + n * _HIDDEN,
                transcendentals=0,
                bytes_accessed=(n * s + n) * _HIDDEN * itemsize),
        )(x)

    tn = sub
    ts = (in_budget // (2 * tn * row_bytes)) // sub * sub
    ts = max(sub, min(ts, _round_up(s, sub)))
    grid = (_cdiv(n, tn), _cdiv(s, ts))
    return pl.pallas_call(
        functools.partial(_pool_streamed_kernel, inv_s=1.0 / s, s=s, ts=ts),
        out_shape=jax.ShapeDtypeStruct((n, _HIDDEN), dtype),
        grid_spec=pltpu.PrefetchScalarGridSpec(
            num_scalar_prefetch=0,
            grid=grid,
            in_specs=[pl.BlockSpec((tn, ts, _HIDDEN), lambda i, k: (i, k, 0))],
            out_specs=pl.BlockSpec((tn, _HIDDEN), lambda i, k: (i, 0)),
            scratch_shapes=[pltpu.VMEM((tn, _HIDDEN), jnp.float32)],
        ),
        compiler_params=pltpu.CompilerParams(
            dimension_semantics=("parallel", "arbitrary"),
            vmem_limit_bytes=64 << 20),
        cost_estimate=pl.CostEstimate(
            flops=n * s * _HIDDEN + n * _HIDDEN,
            transcendentals=0,
            bytes_accessed=(n * s + n) * _HIDDEN * itemsize),
    )(x)
